# baseline (speedup 1.0000x reference)
import jax
import jax.numpy as jnp
from jax import lax
from jax.experimental import pallas as pl
from jax.experimental.pallas import tpu as pltpu
from jax.experimental.pallas import tpu_sc as plsc

N_NODES = 10000
N_EDGES = 160000
D_FEAT = 256
N_REL = 16
H = D_FEAT // 2
NS = 16
L = 16
K = 64
NCHUNKS = N_EDGES // K
NBASE = NCHUNKS // NS
NEXTRA = NCHUNKS - NBASE * NS
NP = 10240
ROWS_PER_TILE = NP // NS
ZR = 32
UNROLL = 12


def _sc_body(af, tail, head, etype, rel2, out,
             acc, rows0, rows1, rows2, relr0, relr1, relv,
             tv0, tv1, hv0, hv1, hv2, hv3, ev0, ev1, zbuf,
             sem_t0, sem_t1, sem_h0, sem_h1, sem_e0, sem_e1,
             sem_g0, sem_g1, sem_r0, sem_r1, sem_s0, sem_s1):
    c = lax.axis_index("c")
    s = lax.axis_index("s")
    rows = (rows0, rows1, rows2)
    relr = (relr0, relr1)
    tv = (tv0, tv1)
    hv = (hv0, hv1, hv2, hv3)
    ev = (ev0, ev1)
    sem_t = (sem_t0, sem_t1)
    sem_h = (sem_h0, sem_h1)
    sem_e = (sem_e0, sem_e1)
    sem_g = (sem_g0, sem_g1)
    sem_r = (sem_r0, sem_r1)
    sem_s = (sem_s0, sem_s1)

    @pl.when(s == 0)
    def _():
        pltpu.sync_copy(rel2.at[pl.ds(c * N_REL, N_REL)], relv)

    zero = jnp.zeros((L,), jnp.float32)
    for i in range(ZR):
        for j in range(H // L):
            zbuf[i, pl.ds(j * L, L)] = zero
    r0 = s * ROWS_PER_TILE

    def _zero_start(i, _):
        pltpu.async_copy(zbuf, acc.at[pl.ds(r0 + i * ZR, ZR)], sem_s0)
        return 0

    def _zero_wait(i, _):
        pltpu.make_async_copy(zbuf, acc.at[pl.ds(r0 + i * ZR, ZR)],
                              sem_s0).wait()
        return 0

    lax.fori_loop(0, ROWS_PER_TILE // ZR, _zero_start, 0)
    lax.fori_loop(0, ROWS_PER_TILE // ZR, _zero_wait, 0)
    plsc.subcore_barrier()

    nt = NBASE + jnp.where(s < NEXTRA, 1, 0)
    ebase = (s * NBASE + jnp.minimum(s, NEXTRA)) * K
    ch = pl.multiple_of(c * H, H)

    def _start_idx(i, u):
        off = ebase + i * K
        pltpu.async_copy(tail.at[pl.ds(off, K)], tv[u % 2], sem_t[u % 2])
        pltpu.async_copy(head.at[pl.ds(off, K)], hv[u % 4], sem_h[u % 2])
        pltpu.async_copy(etype.at[pl.ds(off, K)], ev[u % 2], sem_e[u % 2])

    def _wait_idx(i, u):
        off = ebase + i * K
        pltpu.make_async_copy(
            tail.at[pl.ds(off, K)], tv[u % 2], sem_t[u % 2]).wait()
        pltpu.make_async_copy(
            head.at[pl.ds(off, K)], hv[u % 4], sem_h[u % 2]).wait()
        pltpu.make_async_copy(
            etype.at[pl.ds(off, K)], ev[u % 2], sem_e[u % 2]).wait()

    def _start_gathers(u):
        pltpu.async_copy(af.at[tv[u % 2], pl.ds(ch, H)], rows[u % 3],
                         sem_g[u % 2])
        pltpu.async_copy(relv.at[ev[u % 2]], relr[u % 2], sem_r[u % 2])

    def _wait_gathers(u):
        pltpu.make_async_copy(af.at[tv[u % 2], pl.ds(ch, H)], rows[u % 3],
                          sem_g[u % 2]).wait()
        pltpu.make_async_copy(
            relv.at[ev[u % 2]], relr[u % 2], sem_r[u % 2]).wait()

    def _compute(u):
        def _group(g, _):
            for e in range(8):
                for j in range(H // L):
                    sl = pl.ds(j * L, L)
                    rows[u % 3][g * 8 + e, sl] = (
                        rows[u % 3][g * 8 + e, sl]
                        * relr[u % 2][g * 8 + e, sl])
            return 0

        lax.fori_loop(0, K // 8, _group, 0)

    def _start_scatter(u):
        pltpu.async_copy(rows[u % 3], acc.at[hv[u % 4]], sem_s[u % 2], add=True)

    def _wait_scatter(u):
        pltpu.make_async_copy(
            rows[u % 3], acc.at[hv[u % 4]], sem_s[u % 2]).wait()

    _start_idx(0, 0)
    _start_idx(1, 1)
    _wait_idx(0, 0)
    _start_gathers(0)

    def _iter(i, u):
        @pl.when(i >= 2)
        def _():
            _wait_scatter(u - 2)

        @pl.when(i + 1 < nt)
        def _():
            _wait_idx(i + 1, u + 1)
            _start_gathers(u + 1)

        _wait_gathers(u)

        @pl.when(i + 2 < nt)
        def _():
            _start_idx(i + 2, u + 2)

        _compute(u)
        _start_scatter(u)

    def _twelve(k, _):
        for u in range(UNROLL):
            _iter(k * UNROLL + u, u)
        return 0

    lax.fori_loop(0, NBASE // UNROLL, _twelve, 0)

    @pl.when(s < NEXTRA)
    def _():
        _iter(NBASE, NBASE % UNROLL)
        _wait_scatter(NBASE - 1)
        _wait_scatter(NBASE)

    @pl.when(s >= NEXTRA)
    def _():
        _wait_scatter(NBASE - 2)
        _wait_scatter(NBASE - 1)

    plsc.subcore_barrier()

    @pl.when(s < NS - 1)
    def _():
        pltpu.sync_copy(acc.at[pl.ds(r0, ROWS_PER_TILE)],
                        out.at[pl.ds(r0, ROWS_PER_TILE), pl.ds(ch, H)])

    @pl.when(s == NS - 1)
    def _():
        pltpu.sync_copy(acc.at[pl.ds(r0, N_NODES - (NS - 1) * ROWS_PER_TILE)],
                        out.at[pl.ds(r0, N_NODES - (NS - 1) * ROWS_PER_TILE),
                               pl.ds(ch, H)])


@jax.jit
def _aggregate(af, tail, head, etype, rel2):
    mesh = plsc.VectorSubcoreMesh(core_axis_name="c", subcore_axis_name="s")
    kfn = pl.kernel(
        _sc_body,
        out_type=jax.ShapeDtypeStruct((N_NODES, D_FEAT), jnp.float32),
        mesh=mesh,
        scratch_types=[
            pltpu.VMEM_SHARED((NP, H), jnp.float32),
            pltpu.VMEM((K, H), jnp.float32),
            pltpu.VMEM((K, H), jnp.float32),
            pltpu.VMEM((K, H), jnp.float32),
            pltpu.VMEM((K, H), jnp.float32),
            pltpu.VMEM((K, H), jnp.float32),
            pltpu.VMEM_SHARED((N_REL, H), jnp.float32),
            pltpu.VMEM((K,), jnp.int32),
            pltpu.VMEM((K,), jnp.int32),
            pltpu.VMEM((K,), jnp.int32),
            pltpu.VMEM((K,), jnp.int32),
            pltpu.VMEM((K,), jnp.int32),
            pltpu.VMEM((K,), jnp.int32),
            pltpu.VMEM((K,), jnp.int32),
            pltpu.VMEM((K,), jnp.int32),
            pltpu.VMEM((ZR, H), jnp.float32),
            pltpu.SemaphoreType.DMA,
            pltpu.SemaphoreType.DMA,
            pltpu.SemaphoreType.DMA,
            pltpu.SemaphoreType.DMA,
            pltpu.SemaphoreType.DMA,
            pltpu.SemaphoreType.DMA,
            pltpu.SemaphoreType.DMA,
            pltpu.SemaphoreType.DMA,
            pltpu.SemaphoreType.DMA,
            pltpu.SemaphoreType.DMA,
            pltpu.SemaphoreType.DMA,
            pltpu.SemaphoreType.DMA,
        ],
    )
    return kfn(af, tail, head, etype, rel2)


def kernel(all_emb, edge_index, edge_type, weight, relation_emb):
    del weight
    rel2 = jnp.concatenate([relation_emb[:, :H], relation_emb[:, H:]], axis=0)
    return _aggregate(all_emb, edge_index[1], edge_index[0], edge_type, rel2)


# speedup vs baseline: 1.0679x; 1.0679x over previous
import jax
import jax.numpy as jnp
from jax import lax
from jax.experimental import pallas as pl
from jax.experimental.pallas import tpu as pltpu
from jax.experimental.pallas import tpu_sc as plsc

N_NODES = 10000
N_EDGES = 160000
D_FEAT = 256
N_REL = 16
H = D_FEAT // 2
NS = 16
L = 16
K = 64
NCHUNKS = N_EDGES // K
NBASE = NCHUNKS // NS
NEXTRA = NCHUNKS - NBASE * NS
NP = 10240
ROWS_PER_TILE = NP // NS
ZR = 32
UNROLL = 12


def _sc_body(a2, tail, head, etype, rel2, out,
             acc, rows0, rows1, rows2, relr0, relr1, relv,
             tv0, tv1, hv0, hv1, hv2, hv3, ev0, ev1, zbuf,
             sem_t0, sem_t1, sem_h0, sem_h1, sem_e0, sem_e1,
             sem_g0, sem_g1, sem_r0, sem_r1, sem_s0, sem_s1):
    c = lax.axis_index("c")
    s = lax.axis_index("s")
    rows = (rows0, rows1, rows2)
    relr = (relr0, relr1)
    tv = (tv0, tv1)
    hv = (hv0, hv1, hv2, hv3)
    ev = (ev0, ev1)
    sem_t = (sem_t0, sem_t1)
    sem_h = (sem_h0, sem_h1)
    sem_e = (sem_e0, sem_e1)
    sem_g = (sem_g0, sem_g1)
    sem_r = (sem_r0, sem_r1)
    sem_s = (sem_s0, sem_s1)

    @pl.when(s == 0)
    def _():
        pltpu.sync_copy(rel2.at[pl.ds(c * N_REL, N_REL)], relv)

    zero = jnp.zeros((L,), jnp.float32)
    for i in range(ZR):
        for j in range(H // L):
            zbuf[i, pl.ds(j * L, L)] = zero
    r0 = s * ROWS_PER_TILE

    def _zero_start(i, _):
        pltpu.async_copy(zbuf, acc.at[pl.ds(r0 + i * ZR, ZR)], sem_s0)
        return 0

    def _zero_wait(i, _):
        pltpu.make_async_copy(zbuf, acc.at[pl.ds(r0 + i * ZR, ZR)],
                              sem_s0).wait()
        return 0

    lax.fori_loop(0, ROWS_PER_TILE // ZR, _zero_start, 0)
    lax.fori_loop(0, ROWS_PER_TILE // ZR, _zero_wait, 0)
    plsc.subcore_barrier()

    nt = NBASE + jnp.where(s < NEXTRA, 1, 0)
    ebase = (s * NBASE + jnp.minimum(s, NEXTRA)) * K
    ch = pl.multiple_of(c * H, H)
    cofs_n = c * N_NODES

    def _start_idx(i, u):
        off = ebase + i * K
        pltpu.async_copy(tail.at[pl.ds(off, K)], tv[u % 2], sem_t[u % 2])
        pltpu.async_copy(head.at[pl.ds(off, K)], hv[u % 4], sem_h[u % 2])
        pltpu.async_copy(etype.at[pl.ds(off, K)], ev[u % 2], sem_e[u % 2])

    def _wait_idx(i, u):
        off = ebase + i * K
        pltpu.make_async_copy(
            tail.at[pl.ds(off, K)], tv[u % 2], sem_t[u % 2]).wait()
        pltpu.make_async_copy(
            head.at[pl.ds(off, K)], hv[u % 4], sem_h[u % 2]).wait()
        pltpu.make_async_copy(
            etype.at[pl.ds(off, K)], ev[u % 2], sem_e[u % 2]).wait()
        for j in range(K // L):
            sl = pl.ds(j * L, L)
            tv[u % 2][sl] = tv[u % 2][sl] + cofs_n

    def _start_gathers(u):
        pltpu.async_copy(a2.at[tv[u % 2]], rows[u % 3], sem_g[u % 2])
        pltpu.async_copy(relv.at[ev[u % 2]], relr[u % 2], sem_r[u % 2])

    def _wait_gathers(u):
        pltpu.make_async_copy(a2.at[tv[u % 2]], rows[u % 3], sem_g[u % 2]).wait()
        pltpu.make_async_copy(
            relv.at[ev[u % 2]], relr[u % 2], sem_r[u % 2]).wait()

    def _compute(u):
        def _group(g, _):
            for e in range(8):
                for j in range(H // L):
                    sl = pl.ds(j * L, L)
                    rows[u % 3][g * 8 + e, sl] = (
                        rows[u % 3][g * 8 + e, sl]
                        * relr[u % 2][g * 8 + e, sl])
            return 0

        lax.fori_loop(0, K // 8, _group, 0)

    def _start_scatter(u):
        pltpu.async_copy(rows[u % 3], acc.at[hv[u % 4]], sem_s[u % 2], add=True)

    def _wait_scatter(u):
        pltpu.make_async_copy(
            rows[u % 3], acc.at[hv[u % 4]], sem_s[u % 2]).wait()

    _start_idx(0, 0)
    _start_idx(1, 1)
    _wait_idx(0, 0)
    _start_gathers(0)

    def _iter(i, u):
        @pl.when(i >= 2)
        def _():
            _wait_scatter(u - 2)

        @pl.when(i + 1 < nt)
        def _():
            _wait_idx(i + 1, u + 1)
            _start_gathers(u + 1)

        _wait_gathers(u)

        @pl.when(i + 2 < nt)
        def _():
            _start_idx(i + 2, u + 2)

        _compute(u)
        _start_scatter(u)

    def _twelve(k, _):
        for u in range(UNROLL):
            _iter(k * UNROLL + u, u)
        return 0

    lax.fori_loop(0, NBASE // UNROLL, _twelve, 0)

    @pl.when(s < NEXTRA)
    def _():
        _iter(NBASE, NBASE % UNROLL)
        _wait_scatter(NBASE - 1)
        _wait_scatter(NBASE)

    @pl.when(s >= NEXTRA)
    def _():
        _wait_scatter(NBASE - 2)
        _wait_scatter(NBASE - 1)

    plsc.subcore_barrier()

    @pl.when(s < NS - 1)
    def _():
        pltpu.sync_copy(acc.at[pl.ds(r0, ROWS_PER_TILE)],
                        out.at[pl.ds(r0, ROWS_PER_TILE), pl.ds(ch, H)])

    @pl.when(s == NS - 1)
    def _():
        pltpu.sync_copy(acc.at[pl.ds(r0, N_NODES - (NS - 1) * ROWS_PER_TILE)],
                        out.at[pl.ds(r0, N_NODES - (NS - 1) * ROWS_PER_TILE),
                               pl.ds(ch, H)])


@jax.jit
def _aggregate(a2, tail, head, etype, rel2):
    mesh = plsc.VectorSubcoreMesh(core_axis_name="c", subcore_axis_name="s")
    kfn = pl.kernel(
        _sc_body,
        out_type=jax.ShapeDtypeStruct((N_NODES, D_FEAT), jnp.float32),
        mesh=mesh,
        scratch_types=[
            pltpu.VMEM_SHARED((NP, H), jnp.float32),
            pltpu.VMEM((K, H), jnp.float32),
            pltpu.VMEM((K, H), jnp.float32),
            pltpu.VMEM((K, H), jnp.float32),
            pltpu.VMEM((K, H), jnp.float32),
            pltpu.VMEM((K, H), jnp.float32),
            pltpu.VMEM_SHARED((N_REL, H), jnp.float32),
            pltpu.VMEM((K,), jnp.int32),
            pltpu.VMEM((K,), jnp.int32),
            pltpu.VMEM((K,), jnp.int32),
            pltpu.VMEM((K,), jnp.int32),
            pltpu.VMEM((K,), jnp.int32),
            pltpu.VMEM((K,), jnp.int32),
            pltpu.VMEM((K,), jnp.int32),
            pltpu.VMEM((K,), jnp.int32),
            pltpu.VMEM((ZR, H), jnp.float32),
            pltpu.SemaphoreType.DMA,
            pltpu.SemaphoreType.DMA,
            pltpu.SemaphoreType.DMA,
            pltpu.SemaphoreType.DMA,
            pltpu.SemaphoreType.DMA,
            pltpu.SemaphoreType.DMA,
            pltpu.SemaphoreType.DMA,
            pltpu.SemaphoreType.DMA,
            pltpu.SemaphoreType.DMA,
            pltpu.SemaphoreType.DMA,
            pltpu.SemaphoreType.DMA,
            pltpu.SemaphoreType.DMA,
        ],
    )
    return kfn(a2, tail, head, etype, rel2)


def kernel(all_emb, edge_index, edge_type, weight, relation_emb):
    del weight
    a2 = jnp.concatenate([all_emb[:, :H], all_emb[:, H:]], axis=0)
    rel2 = jnp.concatenate([relation_emb[:, :H], relation_emb[:, H:]], axis=0)
    return _aggregate(a2, edge_index[1], edge_index[0], edge_type, rel2)
